# Initial kernel scaffold; baseline (speedup 1.0000x reference)
import jax
import jax.numpy as jnp
from jax.experimental import pallas as pl


def kernel(Ahat, node_embs, word_level_features, mask, ht, Wm, bm, Wu, Uu, bu, Wr, Ur, br, Wh, Uh, bh, evolve_A, GCN_init_mapping, Wi, bi):
    raise NotImplementedError("write your pallas kernel here")


# speedup vs baseline: 2.7417x; 2.7417x over previous
import functools

import jax
import jax.numpy as jnp
from jax import lax
from jax.experimental import pallas as pl
from jax.experimental.pallas import tpu as pltpu
from jax.experimental.pallas import tpu_sc as plsc

B, N, L, D, K, R = 8, 2048, 20, 256, 128, 512
NC, NS = 2, 16
NW = NC * NS
RPW = (B * K) // NW
WPB = NW // B


def _k1_body(ne_ref, ht_ref, wm_ref, bm_ref, wi_ref, bi_ref,
             scores_ref, scorer_ref, ht2_ref, stats_ref):
    ht_b = ht_ref[0]
    f32, bf16 = jnp.float32, jnp.bfloat16
    dn = (((1,), (1,)), ((), ()))

    def _lo(x):
        return x.astype(bf16).astype(f32)

    ht_lo = _lo(ht_b)
    scorer = jnp.tanh(
        lax.dot_general(ht_lo, _lo(wm_ref[...]), dn, preferred_element_type=f32)
        + bm_ref[...])
    scorer_ref[0] = scorer
    ht2_ref[0] = jnp.tanh(
        lax.dot_general(ht_lo, _lo(wi_ref[...]), dn, preferred_element_type=f32)
        + bi_ref[...])
    nrm = jnp.sqrt(jnp.sum(scorer * scorer))
    embs = ne_ref[0]
    raw = lax.dot_general(_lo(embs), _lo(scorer), dn,
                          preferred_element_type=f32)
    s = raw[:, 0] / nrm
    scores_ref[0, 0] = s
    m = jnp.max(s)
    e = jnp.exp(s - m)
    z = jnp.sum(e)
    logz = jnp.log(z) + m
    ent = logz - jnp.sum(e * s) / z
    li = lax.broadcasted_iota(jnp.int32, (1, 1, K), 2)
    stats_ref[...] = jnp.where(li == 0, logz, jnp.where(li == 1, ent, 0.0))


def _run_k1(node_embs, ht3, Wm, bm2, Wi, bi2):
    return pl.pallas_call(
        _k1_body,
        grid=(B,),
        in_specs=[
            pl.BlockSpec((1, N, D), lambda b: (b, 0, 0)),
            pl.BlockSpec((1, 1, R), lambda b: (b, 0, 0)),
            pl.BlockSpec((D, R), lambda b: (0, 0)),
            pl.BlockSpec((1, D), lambda b: (0, 0)),
            pl.BlockSpec((D, R), lambda b: (0, 0)),
            pl.BlockSpec((1, D), lambda b: (0, 0)),
        ],
        out_specs=[
            pl.BlockSpec((1, 1, N), lambda b: (b, 0, 0)),
            pl.BlockSpec((1, 1, D), lambda b: (b, 0, 0)),
            pl.BlockSpec((1, 1, D), lambda b: (b, 0, 0)),
            pl.BlockSpec((1, 1, K), lambda b: (b, 0, 0)),
        ],
        out_shape=[
            jax.ShapeDtypeStruct((B, 1, N), jnp.float32),
            jax.ShapeDtypeStruct((B, 1, D), jnp.float32),
            jax.ShapeDtypeStruct((B, 1, D), jnp.float32),
            jax.ShapeDtypeStruct((B, 1, K), jnp.float32),
        ],
    )(node_embs, ht3, Wm, bm2, Wi, bi2)


def _k2_body(scores_ref, stats_ref, vals_ref, idx_ref, gidx_ref, pol_ref):
    iota_n = lax.broadcasted_iota(jnp.int32, (B, N), 1)
    iota_k = lax.broadcasted_iota(jnp.int32, (B, K), 1)

    def body(it, c):
        s, va, ia = c
        m = jnp.max(s, axis=1, keepdims=True)
        sel = jnp.min(jnp.where(s == m, iota_n, N), axis=1, keepdims=True)
        va = jnp.where(iota_k == it, m, va)
        ia = jnp.where(iota_k == it, sel, ia)
        s = jnp.where(iota_n == sel, -jnp.inf, s)
        return s, va, ia

    s0 = scores_ref[...]
    _, vals, idxs = lax.fori_loop(
        0, K, body,
        (s0, jnp.zeros((B, K), jnp.float32), jnp.zeros((B, K), jnp.int32)))
    vals_ref[...] = vals
    idx_ref[...] = idxs
    rowoff = lax.broadcasted_iota(jnp.int32, (B, K), 0) * N
    gidx_ref[...] = idxs + rowoff
    logz = stats_ref[:, 0:1]
    pol = jnp.mean(vals, axis=1, keepdims=True) - logz
    pol_ref[...] = jnp.broadcast_to(pol, (B, K))


def _run_k2(scores, stats):
    return pl.pallas_call(
        _k2_body,
        out_shape=[
            jax.ShapeDtypeStruct((B, K), jnp.float32),
            jax.ShapeDtypeStruct((B, K), jnp.int32),
            jax.ShapeDtypeStruct((B, K), jnp.int32),
            jax.ShapeDtypeStruct((B, K), jnp.float32),
        ],
    )(scores, stats)


def _k3_body(gidx_hbm, idx_hbm, nef_hbm, ahf_hbm, ne_out, ah_out,
             ridx_v, cidx_v, ne_v, ar_v, ah_v, sem1, sem2):
    wid = lax.axis_index("s") * NC + lax.axis_index("c")
    b = wid // WPB
    base = b * K + (wid % WPB) * RPW
    pltpu.sync_copy(gidx_hbm.at[pl.ds(base, RPW)], ridx_v)
    pltpu.sync_copy(idx_hbm.at[pl.ds(b * K, K)], cidx_v)
    cp1 = pltpu.async_copy(nef_hbm.at[ridx_v], ne_v, sem1)
    cp2 = pltpu.async_copy(ahf_hbm.at[ridx_v], ar_v, sem2)
    cp1.wait()
    pltpu.sync_copy(ne_v, ne_out.at[pl.ds(base, RPW)])
    cp2.wait()

    def row(r, _):
        for j in range(K // 16):
            row16 = jnp.zeros((16,), jnp.int32) + r
            col16 = cidx_v[pl.ds(j * 16, 16)]
            g = plsc.load_gather(ar_v, [row16, col16])
            ah_v[pl.ds(r * K + j * 16, 16)] = g
        return 0

    lax.fori_loop(0, RPW, row, 0)
    pltpu.sync_copy(ah_v, ah_out.at[pl.ds(base * K, RPW * K)])


@functools.lru_cache(maxsize=1)
def _k3_call():
    return pl.kernel(
        _k3_body,
        out_type=(
            jax.ShapeDtypeStruct((B * K, D), jnp.float32),
            jax.ShapeDtypeStruct((B * K * K,), jnp.float32),
        ),
        mesh=plsc.VectorSubcoreMesh(core_axis_name="c", subcore_axis_name="s"),
        compiler_params=pltpu.CompilerParams(needs_layout_passes=False),
        scratch_types=[
            pltpu.VMEM((RPW,), jnp.int32),
            pltpu.VMEM((K,), jnp.int32),
            pltpu.VMEM((RPW, D), jnp.float32),
            pltpu.VMEM((RPW, N), jnp.float32),
            pltpu.VMEM((RPW * K,), jnp.float32),
            pltpu.SemaphoreType.DMA,
            pltpu.SemaphoreType.DMA,
        ],
    )


def _run_k3(gidx_flat, idx_flat, nef, ahf):
    return _k3_call()(gidx_flat, idx_flat, nef, ahf)


def _k4_body(ne_ref, ah_ref, vals_ref, wlf_ref, ht2_ref,
             wu_ref, uu_ref, bu_ref, wr_ref, ur_ref, br_ref,
             wh_ref, uh_ref, bh_ref, ev_ref, gim_ref, out_ref):
    f32 = jnp.float32
    dt = (((1,), (1,)), ((), ()))
    dm = (((1,), (0,)), ((), ()))
    ne = ne_ref[0]
    z = ne[:, :K]
    tv = jnp.tanh(vals_ref[0])
    ev = ev_ref[...]
    cu = lax.dot_general(uu_ref[...], ev, dm, preferred_element_type=f32) + bu_ref[...]
    cr = lax.dot_general(ur_ref[...], ev, dm, preferred_element_type=f32) + br_ref[...]
    pu = lax.dot_general(wu_ref[...], z, dt, preferred_element_type=f32) * tv
    pr = lax.dot_general(wr_ref[...], z, dt, preferred_element_type=f32) * tv
    ph = lax.dot_general(wh_ref[...], z, dt, preferred_element_type=f32) * tv
    upd = jax.nn.sigmoid(pu + cu)
    rst = jax.nn.sigmoid(pr + cr)
    hc0 = rst * ev
    hcap = jnp.tanh(ph + lax.dot_general(uh_ref[...], hc0, dm, preferred_element_type=f32)
                    + bh_ref[...])
    new_q = (1.0 - upd) * ev + upd * hcap
    wl = wlf_ref[0]
    at = lax.dot_general(ne, wl, dt, preferred_element_type=f32)
    attn_a = lax.dot_general(at, at, dt, preferred_element_type=f32)
    ah = ah_ref[0]
    di = lax.rsqrt(jnp.sum(ah, axis=0))
    ahn = ah * di[None, :] * di[:, None]
    t_a = new_q + ahn + attn_a
    v = jnp.sum(ne * ht2_ref[0], axis=1)
    u = jnp.sum(t_a * v[None, :], axis=1)
    out_ref[0] = jnp.maximum(u[:, None] * gim_ref[...], 0.0)


def _run_k4(ne4, ah4, vals3, wlf, ht23, Wu, Uu, bu, Wr, Ur, br, Wh, Uh, bh,
            evolve_A, gim):
    full_kk = pl.BlockSpec((K, K), lambda b: (0, 0))
    return pl.pallas_call(
        _k4_body,
        grid=(B,),
        in_specs=[
            pl.BlockSpec((1, K, D), lambda b: (b, 0, 0)),
            pl.BlockSpec((1, K, K), lambda b: (b, 0, 0)),
            pl.BlockSpec((1, 1, K), lambda b: (b, 0, 0)),
            pl.BlockSpec((1, L, D), lambda b: (b, 0, 0)),
            pl.BlockSpec((1, 1, D), lambda b: (b, 0, 0)),
            full_kk, full_kk, full_kk, full_kk, full_kk,
            full_kk, full_kk, full_kk, full_kk, full_kk,
            pl.BlockSpec((1, D), lambda b: (0, 0)),
        ],
        out_specs=pl.BlockSpec((1, K, D), lambda b: (b, 0, 0)),
        out_shape=jax.ShapeDtypeStruct((B, K, D), jnp.float32),
    )(ne4, ah4, vals3, wlf, ht23, Wu, Uu, bu, Wr, Ur, br, Wh, Uh, bh,
      evolve_A, gim)


def kernel(Ahat, node_embs, word_level_features, mask, ht, Wm, bm, Wu, Uu, bu,
           Wr, Ur, br, Wh, Uh, bh, evolve_A, GCN_init_mapping, Wi, bi):
    ht3 = ht.reshape(B, 1, R)
    scores3, scorer3, ht23, stats3 = _run_k1(
        node_embs, ht3, Wm, bm.reshape(1, D), Wi, bi.reshape(1, D))
    scores = scores3.reshape(B, N)
    stats = stats3.reshape(B, K)
    vals, idxs, gidx, pol = _run_k2(scores, stats)
    ne_flat, ah_flat = _run_k3(
        gidx.reshape(B * K), idxs.reshape(B * K),
        node_embs.reshape(B * N, D), Ahat.reshape(B * N, N))
    out_embs = _run_k4(
        ne_flat.reshape(B, K, D), ah_flat.reshape(B, K, K),
        vals.reshape(B, 1, K), word_level_features, ht23,
        Wu, Uu, bu, Wr, Ur, br, Wh, Uh, bh, evolve_A, GCN_init_mapping)
    policy_score = pol[:, 0]
    scorer = scorer3.reshape(B, D)
    entropy_object = stats[:, 1]
    return out_embs, policy_score, scorer, entropy_object


# speedup vs baseline: 4.0248x; 1.4680x over previous
import functools

import jax
import jax.numpy as jnp
from jax import lax
from jax.experimental import pallas as pl
from jax.experimental.pallas import tpu as pltpu
from jax.experimental.pallas import tpu_sc as plsc

B, N, L, D, K, R = 8, 2048, 20, 256, 128, 512
NC, NS = 2, 16
NW = NC * NS
RPW = (B * K) // NW
WPB = NW // B


def _k1_body(ne_ref, ht_ref, wm_ref, bm_ref, wi_ref, bi_ref,
             scores_ref, scorer_ref, ht2_ref, stats_ref):
    ht_b = ht_ref[0]
    f32, bf16 = jnp.float32, jnp.bfloat16
    dn = (((1,), (1,)), ((), ()))

    def _lo(x):
        return x.astype(bf16).astype(f32)

    ht_lo = _lo(ht_b)
    scorer = jnp.tanh(
        lax.dot_general(ht_lo, _lo(wm_ref[...]), dn, preferred_element_type=f32)
        + bm_ref[...])
    scorer_ref[0] = scorer
    ht2_ref[0] = jnp.tanh(
        lax.dot_general(ht_lo, _lo(wi_ref[...]), dn, preferred_element_type=f32)
        + bi_ref[...])
    nrm = jnp.sqrt(jnp.sum(scorer * scorer))
    embs = ne_ref[0]
    raw = lax.dot_general(_lo(scorer), _lo(embs), dn,
                          preferred_element_type=f32)
    s = raw[0] / nrm
    scores_ref[0, 0] = s
    m = jnp.max(s)
    e = jnp.exp(s - m)
    z = jnp.sum(e)
    logz = jnp.log(z) + m
    ent = logz - jnp.sum(e * s) / z
    li = lax.broadcasted_iota(jnp.int32, (1, 1, K), 2)
    stats_ref[...] = jnp.where(li == 0, logz, jnp.where(li == 1, ent, 0.0))


def _run_k1(node_embs, ht3, Wm, bm2, Wi, bi2):
    return pl.pallas_call(
        _k1_body,
        grid=(B,),
        in_specs=[
            pl.BlockSpec((1, N, D), lambda b: (b, 0, 0)),
            pl.BlockSpec((1, 1, R), lambda b: (b, 0, 0)),
            pl.BlockSpec((D, R), lambda b: (0, 0)),
            pl.BlockSpec((1, D), lambda b: (0, 0)),
            pl.BlockSpec((D, R), lambda b: (0, 0)),
            pl.BlockSpec((1, D), lambda b: (0, 0)),
        ],
        out_specs=[
            pl.BlockSpec((1, 1, N), lambda b: (b, 0, 0)),
            pl.BlockSpec((1, 1, D), lambda b: (b, 0, 0)),
            pl.BlockSpec((1, 1, D), lambda b: (b, 0, 0)),
            pl.BlockSpec((1, 1, K), lambda b: (b, 0, 0)),
        ],
        out_shape=[
            jax.ShapeDtypeStruct((B, 1, N), jnp.float32),
            jax.ShapeDtypeStruct((B, 1, D), jnp.float32),
            jax.ShapeDtypeStruct((B, 1, D), jnp.float32),
            jax.ShapeDtypeStruct((B, 1, K), jnp.float32),
        ],
    )(node_embs, ht3, Wm, bm2, Wi, bi2)


def _k2_body(scores_ref, stats_ref, vals_ref, idx_ref, gidx_ref, pol_ref):
    lane = lax.broadcasted_iota(jnp.int32, (B, N), 1)
    v = scores_ref[...]
    ix = lane
    for k in range(1, 12):
        up = ((lane >> k) & 1) == 0
        for j in range(k - 1, -1, -1):
            d = 1 << j
            first = (lane & d) == 0
            pv = jnp.where(first, jnp.roll(v, -d, axis=1), jnp.roll(v, d, axis=1))
            pi = jnp.where(first, jnp.roll(ix, -d, axis=1), jnp.roll(ix, d, axis=1))
            mine_wins = (v > pv) | ((v == pv) & (ix < pi))
            keep = mine_wins == (first == up)
            v = jnp.where(keep, v, pv)
            ix = jnp.where(keep, ix, pi)
    vals = v[:, :K]
    idxs = ix[:, :K]
    vals_ref[...] = vals
    idx_ref[...] = idxs
    rowoff = lax.broadcasted_iota(jnp.int32, (B, K), 0) * N
    gidx_ref[...] = idxs + rowoff
    logz = stats_ref[:, 0:1]
    pol = jnp.mean(vals, axis=1, keepdims=True) - logz
    pol_ref[...] = jnp.broadcast_to(pol, (B, K))


def _run_k2(scores, stats):
    return pl.pallas_call(
        _k2_body,
        out_shape=[
            jax.ShapeDtypeStruct((B, K), jnp.float32),
            jax.ShapeDtypeStruct((B, K), jnp.int32),
            jax.ShapeDtypeStruct((B, K), jnp.int32),
            jax.ShapeDtypeStruct((B, K), jnp.float32),
        ],
    )(scores, stats)


_NCHUNK = 4
_CROWS = RPW // _NCHUNK


def _k3_body(gidx_hbm, idx_hbm, nef_hbm, ahf_hbm, ne_out, ah_out,
             ridx_v, cidx_v, ne_v, ar_v, ah_v, semn, *sems):
    wid = lax.axis_index("s") * NC + lax.axis_index("c")
    b = wid // WPB
    base = b * K + (wid % WPB) * RPW
    pltpu.sync_copy(gidx_hbm.at[pl.ds(base, RPW)], ridx_v)
    pltpu.sync_copy(idx_hbm.at[pl.ds(b * K, K)], cidx_v)
    cps = [
        pltpu.async_copy(
            ahf_hbm.at[ridx_v.at[pl.ds(c * _CROWS, _CROWS)]],
            ar_v.at[pl.ds(c * _CROWS, _CROWS)], sems[c])
        for c in range(_NCHUNK)
    ]
    cpn = pltpu.async_copy(nef_hbm.at[ridx_v], ne_v, semn)

    def row(r, _):
        for j in range(K // 16):
            row16 = jnp.zeros((16,), jnp.int32) + r
            col16 = cidx_v[pl.ds(j * 16, 16)]
            g = plsc.load_gather(ar_v, [row16, col16])
            ah_v[pl.ds(r * K + j * 16, 16)] = g
        return 0

    for c in range(_NCHUNK):
        cps[c].wait()
        lax.fori_loop(c * _CROWS, (c + 1) * _CROWS, row, 0)
    cpn.wait()
    pltpu.sync_copy(ne_v, ne_out.at[pl.ds(base, RPW)])
    pltpu.sync_copy(ah_v, ah_out.at[pl.ds(base * K, RPW * K)])


@functools.lru_cache(maxsize=1)
def _k3_call():
    return pl.kernel(
        _k3_body,
        out_type=(
            jax.ShapeDtypeStruct((B * K, D), jnp.float32),
            jax.ShapeDtypeStruct((B * K * K,), jnp.float32),
        ),
        mesh=plsc.VectorSubcoreMesh(core_axis_name="c", subcore_axis_name="s"),
        compiler_params=pltpu.CompilerParams(needs_layout_passes=False),
        scratch_types=[
            pltpu.VMEM((RPW,), jnp.int32),
            pltpu.VMEM((K,), jnp.int32),
            pltpu.VMEM((RPW, D), jnp.float32),
            pltpu.VMEM((RPW, N), jnp.float32),
            pltpu.VMEM((RPW * K,), jnp.float32),
            pltpu.SemaphoreType.DMA,
            pltpu.SemaphoreType.DMA,
            pltpu.SemaphoreType.DMA,
            pltpu.SemaphoreType.DMA,
            pltpu.SemaphoreType.DMA,
        ],
    )


def _run_k3(gidx_flat, idx_flat, nef, ahf):
    return _k3_call()(gidx_flat, idx_flat, nef, ahf)


def _k4_body(ne_ref, ah_ref, vals_ref, wlf_ref, ht2_ref,
             wu_ref, uu_ref, bu_ref, wr_ref, ur_ref, br_ref,
             wh_ref, uh_ref, bh_ref, ev_ref, gim_ref, out_ref):
    f32 = jnp.float32
    dt = (((1,), (1,)), ((), ()))
    dm = (((1,), (0,)), ((), ()))
    ne = ne_ref[0]
    z = ne[:, :K]
    tv = jnp.tanh(vals_ref[0])
    ev = ev_ref[...]
    cu = lax.dot_general(uu_ref[...], ev, dm, preferred_element_type=f32) + bu_ref[...]
    cr = lax.dot_general(ur_ref[...], ev, dm, preferred_element_type=f32) + br_ref[...]
    pu = lax.dot_general(wu_ref[...], z, dt, preferred_element_type=f32) * tv
    pr = lax.dot_general(wr_ref[...], z, dt, preferred_element_type=f32) * tv
    ph = lax.dot_general(wh_ref[...], z, dt, preferred_element_type=f32) * tv
    upd = jax.nn.sigmoid(pu + cu)
    rst = jax.nn.sigmoid(pr + cr)
    hc0 = rst * ev
    hcap = jnp.tanh(ph + lax.dot_general(uh_ref[...], hc0, dm, preferred_element_type=f32)
                    + bh_ref[...])
    new_q = (1.0 - upd) * ev + upd * hcap
    wl = wlf_ref[0]
    at = lax.dot_general(ne, wl, dt, preferred_element_type=f32)
    attn_a = lax.dot_general(at, at, dt, preferred_element_type=f32)
    ah = ah_ref[0]
    di = lax.rsqrt(jnp.sum(ah, axis=0))
    ahn = ah * di[None, :] * di[:, None]
    t_a = new_q + ahn + attn_a
    v = jnp.sum(ne * ht2_ref[0], axis=1)
    u = jnp.sum(t_a * v[None, :], axis=1)
    out_ref[0] = jnp.maximum(u[:, None] * gim_ref[...], 0.0)


def _run_k4(ne4, ah4, vals3, wlf, ht23, Wu, Uu, bu, Wr, Ur, br, Wh, Uh, bh,
            evolve_A, gim):
    full_kk = pl.BlockSpec((K, K), lambda b: (0, 0))
    return pl.pallas_call(
        _k4_body,
        grid=(B,),
        in_specs=[
            pl.BlockSpec((1, K, D), lambda b: (b, 0, 0)),
            pl.BlockSpec((1, K, K), lambda b: (b, 0, 0)),
            pl.BlockSpec((1, 1, K), lambda b: (b, 0, 0)),
            pl.BlockSpec((1, L, D), lambda b: (b, 0, 0)),
            pl.BlockSpec((1, 1, D), lambda b: (b, 0, 0)),
            full_kk, full_kk, full_kk, full_kk, full_kk,
            full_kk, full_kk, full_kk, full_kk, full_kk,
            pl.BlockSpec((1, D), lambda b: (0, 0)),
        ],
        out_specs=pl.BlockSpec((1, K, D), lambda b: (b, 0, 0)),
        out_shape=jax.ShapeDtypeStruct((B, K, D), jnp.float32),
    )(ne4, ah4, vals3, wlf, ht23, Wu, Uu, bu, Wr, Ur, br, Wh, Uh, bh,
      evolve_A, gim)


def kernel(Ahat, node_embs, word_level_features, mask, ht, Wm, bm, Wu, Uu, bu,
           Wr, Ur, br, Wh, Uh, bh, evolve_A, GCN_init_mapping, Wi, bi):
    ht3 = ht.reshape(B, 1, R)
    scores3, scorer3, ht23, stats3 = _run_k1(
        node_embs, ht3, Wm, bm.reshape(1, D), Wi, bi.reshape(1, D))
    scores = scores3.reshape(B, N)
    stats = stats3.reshape(B, K)
    vals, idxs, gidx, pol = _run_k2(scores, stats)
    ne_flat, ah_flat = _run_k3(
        gidx.reshape(B * K), idxs.reshape(B * K),
        node_embs.reshape(B * N, D), Ahat.reshape(B * N, N))
    out_embs = _run_k4(
        ne_flat.reshape(B, K, D), ah_flat.reshape(B, K, K),
        vals.reshape(B, 1, K), word_level_features, ht23,
        Wu, Uu, bu, Wr, Ur, br, Wh, Uh, bh, evolve_A, GCN_init_mapping)
    policy_score = pol[:, 0]
    scorer = scorer3.reshape(B, D)
    entropy_object = stats[:, 1]
    return out_embs, policy_score, scorer, entropy_object


# speedup vs baseline: 4.1293x; 1.0260x over previous
import functools

import jax
import jax.numpy as jnp
from jax import lax
from jax.experimental import pallas as pl
from jax.experimental.pallas import tpu as pltpu
from jax.experimental.pallas import tpu_sc as plsc

B, N, L, D, K, R = 8, 2048, 20, 256, 128, 512
NC, NS = 2, 16
NW = NC * NS
RPW = (B * K) // NW
WPB = NW // B


def _k1_body(ne_ref, ht_ref, wm_ref, bm_ref, wi_ref, bi_ref,
             scores_ref, scorer_ref, ht2_ref, stats_ref):
    ht_b = ht_ref[0]
    f32, bf16 = jnp.float32, jnp.bfloat16
    dn = (((1,), (1,)), ((), ()))

    def _lo(x):
        return x.astype(bf16).astype(f32)

    ht_lo = _lo(ht_b)
    scorer = jnp.tanh(
        lax.dot_general(ht_lo, _lo(wm_ref[...]), dn, preferred_element_type=f32)
        + bm_ref[...])
    scorer_ref[0] = scorer
    ht2_ref[0] = jnp.tanh(
        lax.dot_general(ht_lo, _lo(wi_ref[...]), dn, preferred_element_type=f32)
        + bi_ref[...])
    nrm = jnp.sqrt(jnp.sum(scorer * scorer))
    embs = ne_ref[0]
    raw = lax.dot_general(_lo(scorer), _lo(embs), dn,
                          preferred_element_type=f32)
    s = raw[0] / nrm
    scores_ref[0, 0] = s
    m = jnp.max(s)
    e = jnp.exp(s - m)
    z = jnp.sum(e)
    logz = jnp.log(z) + m
    ent = logz - jnp.sum(e * s) / z
    li = lax.broadcasted_iota(jnp.int32, (1, 1, K), 2)
    stats_ref[...] = jnp.where(li == 0, logz, jnp.where(li == 1, ent, 0.0))


def _run_k1(node_embs, ht3, Wm, bm2, Wi, bi2):
    return pl.pallas_call(
        _k1_body,
        grid=(B,),
        in_specs=[
            pl.BlockSpec((1, N, D), lambda b: (b, 0, 0)),
            pl.BlockSpec((1, 1, R), lambda b: (b, 0, 0)),
            pl.BlockSpec((D, R), lambda b: (0, 0)),
            pl.BlockSpec((1, D), lambda b: (0, 0)),
            pl.BlockSpec((D, R), lambda b: (0, 0)),
            pl.BlockSpec((1, D), lambda b: (0, 0)),
        ],
        out_specs=[
            pl.BlockSpec((1, 1, N), lambda b: (b, 0, 0)),
            pl.BlockSpec((1, 1, D), lambda b: (b, 0, 0)),
            pl.BlockSpec((1, 1, D), lambda b: (b, 0, 0)),
            pl.BlockSpec((1, 1, K), lambda b: (b, 0, 0)),
        ],
        out_shape=[
            jax.ShapeDtypeStruct((B, 1, N), jnp.float32),
            jax.ShapeDtypeStruct((B, 1, D), jnp.float32),
            jax.ShapeDtypeStruct((B, 1, D), jnp.float32),
            jax.ShapeDtypeStruct((B, 1, K), jnp.float32),
        ],
    )(node_embs, ht3, Wm, bm2, Wi, bi2)


def _k2_body(scores_ref, stats_ref, vals_ref, idx_ref, gidx_ref, pol_ref):
    lane = lax.broadcasted_iota(jnp.int32, (B, N), 1)
    v = scores_ref[...]
    ix = lane
    for k in range(1, 12):
        up = ((lane >> k) & 1) == 0
        for j in range(k - 1, -1, -1):
            d = 1 << j
            first = (lane & d) == 0
            pv = jnp.where(first, jnp.roll(v, -d, axis=1), jnp.roll(v, d, axis=1))
            pi = jnp.where(first, jnp.roll(ix, -d, axis=1), jnp.roll(ix, d, axis=1))
            mine_wins = (v > pv) | ((v == pv) & (ix < pi))
            keep = mine_wins == (first == up)
            v = jnp.where(keep, v, pv)
            ix = jnp.where(keep, ix, pi)
    vals = v[:, :K]
    idxs = ix[:, :K]
    vals_ref[...] = vals
    idx_ref[...] = idxs
    rowoff = lax.broadcasted_iota(jnp.int32, (B, K), 0) * N
    gidx_ref[...] = idxs + rowoff
    logz = stats_ref[:, 0:1]
    pol = jnp.mean(vals, axis=1, keepdims=True) - logz
    pol_ref[...] = jnp.broadcast_to(pol, (B, K))


def _run_k2(scores, stats):
    return pl.pallas_call(
        _k2_body,
        out_shape=[
            jax.ShapeDtypeStruct((B, K), jnp.float32),
            jax.ShapeDtypeStruct((B, K), jnp.int32),
            jax.ShapeDtypeStruct((B, K), jnp.int32),
            jax.ShapeDtypeStruct((B, K), jnp.float32),
        ],
    )(scores, stats)


_NCHUNK = 4
_CROWS = RPW // _NCHUNK


def _k3_body(gidx_hbm, idx_hbm, nef_hbm, ahf_hbm, ne_out, ah_out,
             ridx_v, cidx_v, ne_v, ar_v, ah_v, semn, *sems):
    wid = lax.axis_index("s") * NC + lax.axis_index("c")
    b = wid // WPB
    base = b * K + (wid % WPB) * RPW
    pltpu.sync_copy(gidx_hbm.at[pl.ds(base, RPW)], ridx_v)
    pltpu.sync_copy(idx_hbm.at[pl.ds(b * K, K)], cidx_v)
    cps = [
        pltpu.async_copy(
            ahf_hbm.at[ridx_v.at[pl.ds(c * _CROWS, _CROWS)]],
            ar_v.at[pl.ds(c * _CROWS, _CROWS)], sems[c])
        for c in range(_NCHUNK)
    ]
    cpn = pltpu.async_copy(nef_hbm.at[ridx_v], ne_v, semn)

    cols = [cidx_v[pl.ds(j * 16, 16)] for j in range(K // 16)]
    for c in range(_NCHUNK):
        cps[c].wait()
        for r in range(c * _CROWS, (c + 1) * _CROWS):
            row16 = jnp.full((16,), r, jnp.int32)
            for j in range(K // 16):
                g = plsc.load_gather(ar_v, [row16, cols[j]])
                ah_v[pl.ds(r * K + j * 16, 16)] = g
    cpn.wait()
    pltpu.sync_copy(ne_v, ne_out.at[pl.ds(base, RPW)])
    pltpu.sync_copy(ah_v, ah_out.at[pl.ds(base * K, RPW * K)])


@functools.lru_cache(maxsize=1)
def _k3_call():
    return pl.kernel(
        _k3_body,
        out_type=(
            jax.ShapeDtypeStruct((B * K, D), jnp.float32),
            jax.ShapeDtypeStruct((B * K * K,), jnp.float32),
        ),
        mesh=plsc.VectorSubcoreMesh(core_axis_name="c", subcore_axis_name="s"),
        compiler_params=pltpu.CompilerParams(needs_layout_passes=False),
        scratch_types=[
            pltpu.VMEM((RPW,), jnp.int32),
            pltpu.VMEM((K,), jnp.int32),
            pltpu.VMEM((RPW, D), jnp.float32),
            pltpu.VMEM((RPW, N), jnp.float32),
            pltpu.VMEM((RPW * K,), jnp.float32),
            pltpu.SemaphoreType.DMA,
            pltpu.SemaphoreType.DMA,
            pltpu.SemaphoreType.DMA,
            pltpu.SemaphoreType.DMA,
            pltpu.SemaphoreType.DMA,
        ],
    )


def _run_k3(gidx_flat, idx_flat, nef, ahf):
    return _k3_call()(gidx_flat, idx_flat, nef, ahf)


def _k4_body(ne_ref, ah_ref, vals_ref, wlf_ref, ht2_ref,
             wu_ref, uu_ref, bu_ref, wr_ref, ur_ref, br_ref,
             wh_ref, uh_ref, bh_ref, ev_ref, gim_ref, out_ref):
    f32 = jnp.float32
    dt = (((1,), (1,)), ((), ()))
    dm = (((1,), (0,)), ((), ()))
    ne = ne_ref[0]
    z = ne[:, :K]
    tv = jnp.tanh(vals_ref[0])
    ev = ev_ref[...]
    cu = lax.dot_general(uu_ref[...], ev, dm, preferred_element_type=f32) + bu_ref[...]
    cr = lax.dot_general(ur_ref[...], ev, dm, preferred_element_type=f32) + br_ref[...]
    pu = lax.dot_general(wu_ref[...], z, dt, preferred_element_type=f32) * tv
    pr = lax.dot_general(wr_ref[...], z, dt, preferred_element_type=f32) * tv
    ph = lax.dot_general(wh_ref[...], z, dt, preferred_element_type=f32) * tv
    upd = jax.nn.sigmoid(pu + cu)
    rst = jax.nn.sigmoid(pr + cr)
    hc0 = rst * ev
    hcap = jnp.tanh(ph + lax.dot_general(uh_ref[...], hc0, dm, preferred_element_type=f32)
                    + bh_ref[...])
    new_q = (1.0 - upd) * ev + upd * hcap
    wl = wlf_ref[0]
    at = lax.dot_general(ne, wl, dt, preferred_element_type=f32)
    attn_a = lax.dot_general(at, at, dt, preferred_element_type=f32)
    ah = ah_ref[0]
    di = lax.rsqrt(jnp.sum(ah, axis=0))
    ahn = ah * di[None, :] * di[:, None]
    t_a = new_q + ahn + attn_a
    v = jnp.sum(ne * ht2_ref[0], axis=1)
    u = jnp.sum(t_a * v[None, :], axis=1)
    out_ref[0] = jnp.maximum(u[:, None] * gim_ref[...], 0.0)


def _run_k4(ne4, ah4, vals3, wlf, ht23, Wu, Uu, bu, Wr, Ur, br, Wh, Uh, bh,
            evolve_A, gim):
    full_kk = pl.BlockSpec((K, K), lambda b: (0, 0))
    return pl.pallas_call(
        _k4_body,
        grid=(B,),
        in_specs=[
            pl.BlockSpec((1, K, D), lambda b: (b, 0, 0)),
            pl.BlockSpec((1, K, K), lambda b: (b, 0, 0)),
            pl.BlockSpec((1, 1, K), lambda b: (b, 0, 0)),
            pl.BlockSpec((1, L, D), lambda b: (b, 0, 0)),
            pl.BlockSpec((1, 1, D), lambda b: (b, 0, 0)),
            full_kk, full_kk, full_kk, full_kk, full_kk,
            full_kk, full_kk, full_kk, full_kk, full_kk,
            pl.BlockSpec((1, D), lambda b: (0, 0)),
        ],
        out_specs=pl.BlockSpec((1, K, D), lambda b: (b, 0, 0)),
        out_shape=jax.ShapeDtypeStruct((B, K, D), jnp.float32),
    )(ne4, ah4, vals3, wlf, ht23, Wu, Uu, bu, Wr, Ur, br, Wh, Uh, bh,
      evolve_A, gim)


def kernel(Ahat, node_embs, word_level_features, mask, ht, Wm, bm, Wu, Uu, bu,
           Wr, Ur, br, Wh, Uh, bh, evolve_A, GCN_init_mapping, Wi, bi):
    ht3 = ht.reshape(B, 1, R)
    scores3, scorer3, ht23, stats3 = _run_k1(
        node_embs, ht3, Wm, bm.reshape(1, D), Wi, bi.reshape(1, D))
    scores = scores3.reshape(B, N)
    stats = stats3.reshape(B, K)
    vals, idxs, gidx, pol = _run_k2(scores, stats)
    ne_flat, ah_flat = _run_k3(
        gidx.reshape(B * K), idxs.reshape(B * K),
        node_embs.reshape(B * N, D), Ahat.reshape(B * N, N))
    out_embs = _run_k4(
        ne_flat.reshape(B, K, D), ah_flat.reshape(B, K, K),
        vals.reshape(B, 1, K), word_level_features, ht23,
        Wu, Uu, bu, Wr, Ur, br, Wh, Uh, bh, evolve_A, GCN_init_mapping)
    policy_score = pol[:, 0]
    scorer = scorer3.reshape(B, D)
    entropy_object = stats[:, 1]
    return out_embs, policy_score, scorer, entropy_object


# speedup vs baseline: 4.3351x; 1.0498x over previous
import functools

import jax
import jax.numpy as jnp
from jax import lax
from jax.experimental import pallas as pl
from jax.experimental.pallas import tpu as pltpu
from jax.experimental.pallas import tpu_sc as plsc

B, N, L, D, K, R = 8, 2048, 20, 256, 128, 512
NC, NS = 2, 16
NW = NC * NS
RPW = (B * K) // NW
WPB = NW // B


def _k1_body(ne_ref, ht_ref, wm_ref, bm_ref, wi_ref, bi_ref,
             scorer_ref, ht2_ref, ent_ref, vals_ref, idx_ref, gidx_ref,
             pol_ref, sc_ref, lz_ref):
    pid = pl.program_id(0)
    f32, bf16 = jnp.float32, jnp.bfloat16
    dn = (((1,), (1,)), ((), ()))

    def _lo(x):
        return x.astype(bf16).astype(f32)

    @pl.when(pid < B)
    def _scores_phase():
        ht_b = ht_ref[0]
        ht_lo = _lo(ht_b)
        scorer = jnp.tanh(
            lax.dot_general(ht_lo, _lo(wm_ref[...]), dn,
                            preferred_element_type=f32) + bm_ref[...])
        scorer_ref[0] = scorer
        ht2_ref[0] = jnp.tanh(
            lax.dot_general(ht_lo, _lo(wi_ref[...]), dn,
                            preferred_element_type=f32) + bi_ref[...])
        nrm = jnp.sqrt(jnp.sum(scorer * scorer))
        embs = ne_ref[0]
        raw = lax.dot_general(_lo(scorer), _lo(embs), dn,
                              preferred_element_type=f32)
        s = raw[0] / nrm
        sc_ref[pid] = s
        m = jnp.max(s)
        e = jnp.exp(s - m)
        z = jnp.sum(e)
        logz = jnp.log(z) + m
        ent = logz - jnp.sum(e * s) / z
        lz_ref[pid] = jnp.full((K,), logz, f32)
        ent_ref[...] = jnp.full((1, 1, K), ent, f32)

    @pl.when(pid == B)
    def _topk_phase():
        lane = lax.broadcasted_iota(jnp.int32, (B, N), 1)
        v = sc_ref[...]
        ix = lane
        for k in range(1, 12):
            up = ((lane >> k) & 1) == 0
            for j in range(k - 1, -1, -1):
                d = 1 << j
                first = (lane & d) == 0
                pv = jnp.where(first, jnp.roll(v, -d, axis=1),
                               jnp.roll(v, d, axis=1))
                pi = jnp.where(first, jnp.roll(ix, -d, axis=1),
                               jnp.roll(ix, d, axis=1))
                mine_wins = (v > pv) | ((v == pv) & (ix < pi))
                keep = mine_wins == (first == up)
                v = jnp.where(keep, v, pv)
                ix = jnp.where(keep, ix, pi)
        vals = v[:, :K]
        idxs = ix[:, :K]
        vals_ref[...] = vals
        idx_ref[...] = idxs
        rowoff = lax.broadcasted_iota(jnp.int32, (B, K), 0) * N
        gidx_ref[...] = idxs + rowoff
        logz = lz_ref[:, 0:1]
        pol = jnp.mean(vals, axis=1, keepdims=True) - logz
        pol_ref[...] = jnp.broadcast_to(pol, (B, K))


def _run_k1(node_embs, ht3, Wm, bm2, Wi, bi2):
    def capped(b):
        return (jnp.minimum(b, B - 1), 0, 0)

    return pl.pallas_call(
        _k1_body,
        grid=(B + 1,),
        in_specs=[
            pl.BlockSpec((1, N, D), capped),
            pl.BlockSpec((1, 1, R), capped),
            pl.BlockSpec((D, R), lambda b: (0, 0)),
            pl.BlockSpec((1, D), lambda b: (0, 0)),
            pl.BlockSpec((D, R), lambda b: (0, 0)),
            pl.BlockSpec((1, D), lambda b: (0, 0)),
        ],
        out_specs=[
            pl.BlockSpec((1, 1, D), capped),
            pl.BlockSpec((1, 1, D), capped),
            pl.BlockSpec((1, 1, K), capped),
            pl.BlockSpec((B, K), lambda b: (0, 0)),
            pl.BlockSpec((B, K), lambda b: (0, 0)),
            pl.BlockSpec((B, K), lambda b: (0, 0)),
            pl.BlockSpec((B, K), lambda b: (0, 0)),
        ],
        out_shape=[
            jax.ShapeDtypeStruct((B, 1, D), jnp.float32),
            jax.ShapeDtypeStruct((B, 1, D), jnp.float32),
            jax.ShapeDtypeStruct((B, 1, K), jnp.float32),
            jax.ShapeDtypeStruct((B, K), jnp.float32),
            jax.ShapeDtypeStruct((B, K), jnp.int32),
            jax.ShapeDtypeStruct((B, K), jnp.int32),
            jax.ShapeDtypeStruct((B, K), jnp.float32),
        ],
        scratch_shapes=[
            pltpu.VMEM((B, N), jnp.float32),
            pltpu.VMEM((B, K), jnp.float32),
        ],
    )(node_embs, ht3, Wm, bm2, Wi, bi2)


_NCHUNK = 4
_CROWS = RPW // _NCHUNK


def _k3_body(gidx_hbm, idx_hbm, nef_hbm, ahf_hbm, ne_out, ah_out,
             ridx_v, cidx_v, ne_v, ar_v, ah_v, semn, *sems):
    wid = lax.axis_index("s") * NC + lax.axis_index("c")
    b = wid // WPB
    base = b * K + (wid % WPB) * RPW
    pltpu.sync_copy(gidx_hbm.at[pl.ds(base, RPW)], ridx_v)
    pltpu.sync_copy(idx_hbm.at[pl.ds(b * K, K)], cidx_v)
    cps = [
        pltpu.async_copy(
            ahf_hbm.at[ridx_v.at[pl.ds(c * _CROWS, _CROWS)]],
            ar_v.at[pl.ds(c * _CROWS, _CROWS)], sems[c])
        for c in range(_NCHUNK)
    ]
    cpn = pltpu.async_copy(nef_hbm.at[ridx_v], ne_v, semn)

    cols = [cidx_v[pl.ds(j * 16, 16)] for j in range(K // 16)]
    for c in range(_NCHUNK):
        cps[c].wait()
        for r in range(c * _CROWS, (c + 1) * _CROWS):
            row16 = jnp.full((16,), r, jnp.int32)
            for j in range(K // 16):
                g = plsc.load_gather(ar_v, [row16, cols[j]])
                ah_v[pl.ds(r * K + j * 16, 16)] = g
    cpn.wait()
    pltpu.sync_copy(ne_v, ne_out.at[pl.ds(base, RPW)])
    pltpu.sync_copy(ah_v, ah_out.at[pl.ds(base * K, RPW * K)])


@functools.lru_cache(maxsize=1)
def _k3_call():
    return pl.kernel(
        _k3_body,
        out_type=(
            jax.ShapeDtypeStruct((B * K, D), jnp.float32),
            jax.ShapeDtypeStruct((B * K * K,), jnp.float32),
        ),
        mesh=plsc.VectorSubcoreMesh(core_axis_name="c", subcore_axis_name="s"),
        compiler_params=pltpu.CompilerParams(needs_layout_passes=False),
        scratch_types=[
            pltpu.VMEM((RPW,), jnp.int32),
            pltpu.VMEM((K,), jnp.int32),
            pltpu.VMEM((RPW, D), jnp.float32),
            pltpu.VMEM((RPW, N), jnp.float32),
            pltpu.VMEM((RPW * K,), jnp.float32),
            pltpu.SemaphoreType.DMA,
            pltpu.SemaphoreType.DMA,
            pltpu.SemaphoreType.DMA,
            pltpu.SemaphoreType.DMA,
            pltpu.SemaphoreType.DMA,
        ],
    )


def _run_k3(gidx_flat, idx_flat, nef, ahf):
    return _k3_call()(gidx_flat, idx_flat, nef, ahf)


def _k4_body(ne_ref, ah_ref, vals_ref, wlf_ref, ht2_ref,
             wu_ref, uu_ref, bu_ref, wr_ref, ur_ref, br_ref,
             wh_ref, uh_ref, bh_ref, ev_ref, gim_ref, out_ref):
    f32 = jnp.float32
    dt = (((1,), (1,)), ((), ()))
    dm = (((1,), (0,)), ((), ()))
    ne = ne_ref[0]
    z = ne[:, :K]
    tv = jnp.tanh(vals_ref[0])
    ev = ev_ref[...]
    cu = lax.dot_general(uu_ref[...], ev, dm, preferred_element_type=f32) + bu_ref[...]
    cr = lax.dot_general(ur_ref[...], ev, dm, preferred_element_type=f32) + br_ref[...]
    pu = lax.dot_general(wu_ref[...], z, dt, preferred_element_type=f32) * tv
    pr = lax.dot_general(wr_ref[...], z, dt, preferred_element_type=f32) * tv
    ph = lax.dot_general(wh_ref[...], z, dt, preferred_element_type=f32) * tv
    upd = jax.nn.sigmoid(pu + cu)
    rst = jax.nn.sigmoid(pr + cr)
    hc0 = rst * ev
    hcap = jnp.tanh(ph + lax.dot_general(uh_ref[...], hc0, dm, preferred_element_type=f32)
                    + bh_ref[...])
    new_q = (1.0 - upd) * ev + upd * hcap
    wl = wlf_ref[0]
    at = lax.dot_general(ne, wl, dt, preferred_element_type=f32)
    attn_a = lax.dot_general(at, at, dt, preferred_element_type=f32)
    ah = ah_ref[0]
    di = lax.rsqrt(jnp.sum(ah, axis=0))
    ahn = ah * di[None, :] * di[:, None]
    t_a = new_q + ahn + attn_a
    v = jnp.sum(ne * ht2_ref[0], axis=1)
    u = jnp.sum(t_a * v[None, :], axis=1)
    out_ref[0] = jnp.maximum(u[:, None] * gim_ref[...], 0.0)


def _run_k4(ne4, ah4, vals3, wlf, ht23, Wu, Uu, bu, Wr, Ur, br, Wh, Uh, bh,
            evolve_A, gim):
    full_kk = pl.BlockSpec((K, K), lambda b: (0, 0))
    return pl.pallas_call(
        _k4_body,
        grid=(B,),
        in_specs=[
            pl.BlockSpec((1, K, D), lambda b: (b, 0, 0)),
            pl.BlockSpec((1, K, K), lambda b: (b, 0, 0)),
            pl.BlockSpec((1, 1, K), lambda b: (b, 0, 0)),
            pl.BlockSpec((1, L, D), lambda b: (b, 0, 0)),
            pl.BlockSpec((1, 1, D), lambda b: (b, 0, 0)),
            full_kk, full_kk, full_kk, full_kk, full_kk,
            full_kk, full_kk, full_kk, full_kk, full_kk,
            pl.BlockSpec((1, D), lambda b: (0, 0)),
        ],
        out_specs=pl.BlockSpec((1, K, D), lambda b: (b, 0, 0)),
        out_shape=jax.ShapeDtypeStruct((B, K, D), jnp.float32),
    )(ne4, ah4, vals3, wlf, ht23, Wu, Uu, bu, Wr, Ur, br, Wh, Uh, bh,
      evolve_A, gim)


def kernel(Ahat, node_embs, word_level_features, mask, ht, Wm, bm, Wu, Uu, bu,
           Wr, Ur, br, Wh, Uh, bh, evolve_A, GCN_init_mapping, Wi, bi):
    ht3 = ht.reshape(B, 1, R)
    scorer3, ht23, ent3, vals, idxs, gidx, pol = _run_k1(
        node_embs, ht3, Wm, bm.reshape(1, D), Wi, bi.reshape(1, D))
    ne_flat, ah_flat = _run_k3(
        gidx.reshape(B * K), idxs.reshape(B * K),
        node_embs.reshape(B * N, D), Ahat.reshape(B * N, N))
    out_embs = _run_k4(
        ne_flat.reshape(B, K, D), ah_flat.reshape(B, K, K),
        vals.reshape(B, 1, K), word_level_features, ht23,
        Wu, Uu, bu, Wr, Ur, br, Wh, Uh, bh, evolve_A, GCN_init_mapping)
    policy_score = pol[:, 0]
    scorer = scorer3.reshape(B, D)
    entropy_object = ent3[:, 0, 0]
    return out_embs, policy_score, scorer, entropy_object
